# baseline (speedup 1.0000x reference)
import functools

import jax
import jax.numpy as jnp
from jax import lax
from jax.experimental import pallas as pl
from jax.experimental.pallas import tpu as pltpu
from jax.experimental.pallas import tpu_sc as plsc

_HEAD = 0
_TAIL = 1
_BSZ, _SEQ, _HID = 128, 512, 1024
_LANES = 16

_NC = 1
_NS = 16
_NW = _NC * _NS
_ROWS_W = _BSZ // _NW
_GATHER_W = 2 * _ROWS_W


def _sc_gather_body(ids_hbm, table_hbm, ent_hbm, ids_v, idx_v, rows_v, sem):
    wid = lax.axis_index("s") * _NC + lax.axis_index("c")
    rbase = wid * _ROWS_W

    pltpu.sync_copy(ids_hbm.at[pl.ds(rbase, _ROWS_W)], ids_v)

    idx_v[...] = jnp.zeros((_LANES,), jnp.int32)
    lane = jax.lax.broadcasted_iota(jnp.int32, (_LANES,), 0)
    base0 = rbase * _SEQ + lane
    for j in range(_ROWS_W):
        base_j = base0 + j * _SEQ

        @plsc.parallel_loop(0, _SEQ, step=_LANES, unroll=4)
        def _scan(p, j=j, base_j=base_j):
            v = ids_v[j, pl.ds(p, _LANES)]
            plsc.store_scatter(idx_v, [v * _ROWS_W + j], base_j + p, mask=v < 2)

    pltpu.async_copy(table_hbm.at[idx_v.at[pl.ds(0, _GATHER_W)]], rows_v, sem).wait()
    pltpu.sync_copy(
        rows_v.at[pl.ds(0, _ROWS_W)],
        ent_hbm.at[pl.ds(rbase, _ROWS_W), pl.ds(0, _HID)],
    )
    pltpu.sync_copy(
        rows_v.at[pl.ds(_ROWS_W, _ROWS_W)],
        ent_hbm.at[pl.ds(rbase, _ROWS_W), pl.ds(_HID, _HID)],
    )


_sc_gather = functools.partial(
    pl.kernel,
    out_type=jax.ShapeDtypeStruct((_BSZ, 2 * _HID), jnp.float32),
    mesh=plsc.VectorSubcoreMesh(core_axis_name="c", subcore_axis_name="s", num_cores=1),
    scratch_types=[
        pltpu.VMEM((_ROWS_W, _SEQ), jnp.int32),
        pltpu.VMEM((_LANES,), jnp.int32),
        pltpu.VMEM((_GATHER_W, _HID), jnp.float32),
        pltpu.SemaphoreType.DMA,
    ],
    compiler_params=pltpu.CompilerParams(needs_layout_passes=False),
)(_sc_gather_body)


def _mm_body(ent_ref, w_ref, b_ref, out_ref):
    out_ref[...] = lax.dot_general(
        ent_ref[...], w_ref[...],
        dimension_numbers=(((1,), (1,)), ((), ())),
        preferred_element_type=jnp.float32,
    ) + b_ref[...]


def kernel(input_ids, last_hidden_state, W, b):
    table = last_hidden_state.reshape(_BSZ * _SEQ, _HID)
    entity = _sc_gather(input_ids, table)
    logits = pl.pallas_call(
        _mm_body,
        out_shape=jax.ShapeDtypeStruct((_BSZ, W.shape[0]), jnp.float32),
    )(entity, W, b.reshape(1, -1))
    return (logits, entity)


# speedup vs baseline: 1.3468x; 1.3468x over previous
import jax
import jax.numpy as jnp
from jax import lax
from jax.experimental import pallas as pl
from jax.experimental.pallas import tpu as pltpu

_HEAD = 0
_TAIL = 1
_BSZ, _SEQ, _HID = 128, 512, 1024
_RPS = 8


def _idx_body(ids_ref, out_ref):
    ids = ids_ref[...]
    col = lax.broadcasted_iota(jnp.int32, (_BSZ, _SEQ), 1)
    out_ref[0, :] = jnp.min(jnp.where(ids == _HEAD, col, _SEQ), axis=1)
    out_ref[1, :] = jnp.min(jnp.where(ids == _TAIL, col, _SEQ), axis=1)


def _gather_mm_body(idx_ref, *refs):
    i = pl.program_id(0)
    hrefs = refs[:_RPS]
    trefs = refs[_RPS:2 * _RPS]
    w_ref, b_ref, ent_ref, log_ref = refs[2 * _RPS:]
    sub = lax.broadcasted_iota(jnp.int32, (8, _HID), 0)
    for j in range(_RPS):
        hp = idx_ref[0, i * _RPS + j]
        tp = idx_ref[1, i * _RPS + j]
        hrow = jnp.sum(jnp.where(sub == hp % 8, hrefs[j][0], 0.0), axis=0)
        trow = jnp.sum(jnp.where(sub == tp % 8, trefs[j][0], 0.0), axis=0)
        ent_ref[j, pl.ds(0, _HID)] = hrow
        ent_ref[j, pl.ds(_HID, _HID)] = trow
    log_ref[...] = lax.dot_general(
        ent_ref[...], w_ref[...],
        dimension_numbers=(((1,), (1,)), ((), ())),
        preferred_element_type=jnp.float32,
    ) + b_ref[...]


def kernel(input_ids, last_hidden_state, W, b):
    nlab = W.shape[0]
    idx = pl.pallas_call(
        _idx_body,
        out_shape=jax.ShapeDtypeStruct((8, _BSZ), jnp.int32),
    )(input_ids)

    def _win_map(j, sel):
        return lambda i, idx_ref: (
            _RPS * i + j, idx_ref[sel, _RPS * i + j] // 8, 0)

    win_spec = lambda m: pl.BlockSpec((1, 8, _HID), m)
    grid_spec = pltpu.PrefetchScalarGridSpec(
        num_scalar_prefetch=1,
        grid=(_BSZ // _RPS,),
        in_specs=[
            *[win_spec(_win_map(j, 0)) for j in range(_RPS)],
            *[win_spec(_win_map(j, 1)) for j in range(_RPS)],
            pl.BlockSpec((nlab, 2 * _HID), lambda i, idx_ref: (0, 0)),
            pl.BlockSpec((1, nlab), lambda i, idx_ref: (0, 0)),
        ],
        out_specs=[
            pl.BlockSpec((_RPS, 2 * _HID), lambda i, idx_ref: (i, 0)),
            pl.BlockSpec((_RPS, nlab), lambda i, idx_ref: (i, 0)),
        ],
    )
    lhs_args = [last_hidden_state] * (2 * _RPS)
    entity, logits = pl.pallas_call(
        _gather_mm_body,
        grid_spec=grid_spec,
        out_shape=(
            jax.ShapeDtypeStruct((_BSZ, 2 * _HID), jnp.float32),
            jax.ShapeDtypeStruct((_BSZ, nlab), jnp.float32),
        ),
    )(idx, *lhs_args, W, b.reshape(1, nlab))
    return (logits, entity)


# speedup vs baseline: 3.4198x; 2.5393x over previous
import jax
import jax.numpy as jnp
from jax import lax
from jax.experimental import pallas as pl
from jax.experimental.pallas import tpu as pltpu

_HEAD = 0
_TAIL = 1
_BSZ, _SEQ, _HID = 128, 512, 1024


def _idx_body(ids_ref, out_ref):
    ids = ids_ref[...]
    col = lax.broadcasted_iota(jnp.int32, (_BSZ, _SEQ), 1)
    out_ref[0, :] = jnp.min(jnp.where(ids == _HEAD, col, _SEQ), axis=1)
    out_ref[1, :] = jnp.min(jnp.where(ids == _TAIL, col, _SEQ), axis=1)


def _gather_mm_body(idx_ref, lhs_ref, w_ref, b_ref, ent_ref, log_ref, sem):
    copies = []
    for r in range(_BSZ):
        hp = idx_ref[0, r]
        tp = idx_ref[1, r]
        ch = pltpu.make_async_copy(
            lhs_ref.at[r, pl.ds(hp, 1), :],
            ent_ref.at[pl.ds(r, 1), pl.ds(0, _HID)], sem)
        ct = pltpu.make_async_copy(
            lhs_ref.at[r, pl.ds(tp, 1), :],
            ent_ref.at[pl.ds(r, 1), pl.ds(_HID, _HID)], sem)
        ch.start()
        ct.start()
        copies.append(ch)
        copies.append(ct)
    for c in copies:
        c.wait()
    log_ref[...] = lax.dot_general(
        ent_ref[...], w_ref[...],
        dimension_numbers=(((1,), (1,)), ((), ())),
        preferred_element_type=jnp.float32,
    ) + b_ref[...]


def kernel(input_ids, last_hidden_state, W, b):
    nlab = W.shape[0]
    idx = pl.pallas_call(
        _idx_body,
        out_shape=jax.ShapeDtypeStruct((8, _BSZ), jnp.int32),
    )(input_ids)

    entity, logits = pl.pallas_call(
        _gather_mm_body,
        in_specs=[
            pl.BlockSpec(memory_space=pltpu.SMEM),
            pl.BlockSpec(memory_space=pl.ANY),
            pl.BlockSpec(memory_space=pltpu.VMEM),
            pl.BlockSpec(memory_space=pltpu.VMEM),
        ],
        out_specs=[
            pl.BlockSpec(memory_space=pltpu.VMEM),
            pl.BlockSpec(memory_space=pltpu.VMEM),
        ],
        out_shape=(
            jax.ShapeDtypeStruct((_BSZ, 2 * _HID), jnp.float32),
            jax.ShapeDtypeStruct((_BSZ, nlab), jnp.float32),
        ),
        scratch_shapes=[pltpu.SemaphoreType.DMA],
    )(idx, last_hidden_state, W, b.reshape(1, nlab))
    return (logits, entity)


# speedup vs baseline: 3.8782x; 1.1341x over previous
import jax
import jax.numpy as jnp
from jax import lax
from jax.experimental import pallas as pl
from jax.experimental.pallas import tpu as pltpu

_HEAD = 0
_TAIL = 1
_BSZ, _SEQ, _HID = 128, 512, 1024


def _fused_body(ids_ref, lhs_ref, w_ref, b_ref, log_ref, ent_ref,
                idx_vmem, idx_smem, sem):
    ids = ids_ref[...]
    col = lax.broadcasted_iota(jnp.int32, (_BSZ, _SEQ), 1)
    idx_vmem[0, :] = jnp.min(jnp.where(ids == _HEAD, col, _SEQ), axis=1)
    idx_vmem[1, :] = jnp.min(jnp.where(ids == _TAIL, col, _SEQ), axis=1)
    bounce = pltpu.make_async_copy(idx_vmem, idx_smem, sem)
    bounce.start()
    bounce.wait()
    copies = []
    for r in range(_BSZ):
        hp = idx_smem[0, r]
        tp = idx_smem[1, r]
        ch = pltpu.make_async_copy(
            lhs_ref.at[r, pl.ds(hp, 1), :],
            ent_ref.at[pl.ds(r, 1), pl.ds(0, _HID)], sem)
        ct = pltpu.make_async_copy(
            lhs_ref.at[r, pl.ds(tp, 1), :],
            ent_ref.at[pl.ds(r, 1), pl.ds(_HID, _HID)], sem)
        ch.start()
        ct.start()
        copies.append(ch)
        copies.append(ct)
    for c in copies:
        c.wait()
    log_ref[...] = lax.dot_general(
        ent_ref[...], w_ref[...],
        dimension_numbers=(((1,), (1,)), ((), ())),
        preferred_element_type=jnp.float32,
    ) + b_ref[...]


def kernel(input_ids, last_hidden_state, W, b):
    nlab = W.shape[0]
    logits, entity = pl.pallas_call(
        _fused_body,
        in_specs=[
            pl.BlockSpec(memory_space=pltpu.VMEM),
            pl.BlockSpec(memory_space=pl.ANY),
            pl.BlockSpec(memory_space=pltpu.VMEM),
            pl.BlockSpec(memory_space=pltpu.VMEM),
        ],
        out_specs=[
            pl.BlockSpec(memory_space=pltpu.VMEM),
            pl.BlockSpec(memory_space=pltpu.VMEM),
        ],
        out_shape=(
            jax.ShapeDtypeStruct((_BSZ, nlab), jnp.float32),
            jax.ShapeDtypeStruct((_BSZ, 2 * _HID), jnp.float32),
        ),
        scratch_shapes=[
            pltpu.VMEM((8, _BSZ), jnp.int32),
            pltpu.SMEM((8, _BSZ), jnp.int32),
            pltpu.SemaphoreType.DMA,
        ],
    )(input_ids, last_hidden_state, W, b.reshape(1, nlab))
    return (logits, entity)


# speedup vs baseline: 4.0446x; 1.0429x over previous
import jax
import jax.numpy as jnp
from jax import lax
from jax.experimental import pallas as pl
from jax.experimental.pallas import tpu as pltpu

_HEAD = 0
_TAIL = 1
_BSZ, _SEQ, _HID = 128, 512, 1024


def _fused_body(ids_ref, lhs_ref, w_ref, b_ref, log_ref, ent_ref,
                idx_vmem, idx_smem, sem):
    ids = ids_ref[...]
    col = lax.broadcasted_iota(jnp.int32, (_BSZ, _SEQ), 1)
    idx_vmem[0, :] = jnp.min(jnp.where(ids == _HEAD, col, _SEQ), axis=1)
    idx_vmem[1, :] = jnp.min(jnp.where(ids == _TAIL, col, _SEQ), axis=1)
    bounce = pltpu.make_async_copy(idx_vmem, idx_smem, sem)
    bounce.start()
    bounce.wait()
    copies = []
    for r in range(_BSZ):
        hp = idx_smem[0, r]
        tp = idx_smem[1, r]
        ch = pltpu.make_async_copy(
            lhs_ref.at[r, pl.ds(hp, 1), :],
            ent_ref.at[pl.ds(r, 1), pl.ds(0, _HID)], sem)
        ct = pltpu.make_async_copy(
            lhs_ref.at[r, pl.ds(tp, 1), :],
            ent_ref.at[pl.ds(r, 1), pl.ds(_HID, _HID)], sem)
        ch.start()
        ct.start()
        copies.append(ch)
        copies.append(ct)
    for c in copies:
        c.wait()
    log_ref[...] = lax.dot_general(
        w_ref[...], ent_ref[...],
        dimension_numbers=(((1,), (1,)), ((), ())),
        preferred_element_type=jnp.float32,
    ) + b_ref[...]


def kernel(input_ids, last_hidden_state, W, b):
    nlab = W.shape[0]
    logits, entity = pl.pallas_call(
        _fused_body,
        in_specs=[
            pl.BlockSpec(memory_space=pltpu.VMEM),
            pl.BlockSpec(memory_space=pl.ANY),
            pl.BlockSpec(memory_space=pltpu.VMEM),
            pl.BlockSpec(memory_space=pltpu.VMEM),
        ],
        out_specs=[
            pl.BlockSpec(memory_space=pltpu.VMEM),
            pl.BlockSpec(memory_space=pltpu.VMEM),
        ],
        out_shape=(
            jax.ShapeDtypeStruct((nlab, _BSZ), jnp.float32),
            jax.ShapeDtypeStruct((_BSZ, 2 * _HID), jnp.float32),
        ),
        scratch_shapes=[
            pltpu.VMEM((8, _BSZ), jnp.int32),
            pltpu.SMEM((8, _BSZ), jnp.int32),
            pltpu.SemaphoreType.DMA,
        ],
    )(input_ids, last_hidden_state, W, b.reshape(nlab, 1))
    return (logits.T, entity)


# speedup vs baseline: 5.0007x; 1.2364x over previous
import jax
import jax.numpy as jnp
from jax import lax
from jax.experimental import pallas as pl
from jax.experimental.pallas import tpu as pltpu

_HEAD = 0
_TAIL = 1
_BSZ, _SEQ, _HID = 128, 512, 1024


def _fused_body(ids_ref, lhs_ref, w_ref, b_ref, log_ref, ent_ref,
                idx_vmem, idx_smem, sem):
    ids = ids_ref[...]
    col = lax.broadcasted_iota(jnp.int32, (_BSZ, _SEQ), 1)
    idx_vmem[0, :] = jnp.min(jnp.where(ids == _HEAD, col, _SEQ), axis=1)
    idx_vmem[1, :] = jnp.min(jnp.where(ids == _TAIL, col, _SEQ), axis=1)
    bounce = pltpu.make_async_copy(idx_vmem, idx_smem, sem)
    bounce.start()
    bounce.wait()
    copies = []
    for r in range(_BSZ):
        hp = idx_smem[0, r]
        tp = idx_smem[1, r]
        ch = pltpu.make_async_copy(
            lhs_ref.at[r, pl.ds(hp, 1), :],
            ent_ref.at[pl.ds(r, 1), pl.ds(0, _HID)], sem)
        ct = pltpu.make_async_copy(
            lhs_ref.at[r, pl.ds(tp, 1), :],
            ent_ref.at[pl.ds(r, 1), pl.ds(_HID, _HID)], sem)
        ch.start()
        ct.start()
        copies.append(ch)
        copies.append(ct)
    for c in copies:
        c.wait()
    log_ref[...] = lax.dot_general(
        w_ref[...], ent_ref[...],
        dimension_numbers=(((1,), (1,)), ((), ())),
        preferred_element_type=jnp.float32,
    ) + jnp.transpose(b_ref[...])


def kernel(input_ids, last_hidden_state, W, b):
    nlab = W.shape[0]
    logits, entity = pl.pallas_call(
        _fused_body,
        in_specs=[
            pl.BlockSpec(memory_space=pltpu.VMEM),
            pl.BlockSpec(memory_space=pl.ANY),
            pl.BlockSpec(memory_space=pltpu.VMEM),
            pl.BlockSpec(memory_space=pltpu.VMEM),
        ],
        out_specs=[
            pl.BlockSpec(memory_space=pltpu.VMEM),
            pl.BlockSpec(memory_space=pltpu.VMEM),
        ],
        out_shape=(
            jax.ShapeDtypeStruct((nlab, _BSZ), jnp.float32),
            jax.ShapeDtypeStruct((_BSZ, 2 * _HID), jnp.float32),
        ),
        scratch_shapes=[
            pltpu.VMEM((8, _BSZ), jnp.int32),
            pltpu.SMEM((8, _BSZ), jnp.int32),
            pltpu.SemaphoreType.DMA,
        ],
    )(input_ids, last_hidden_state, W, b.reshape(1, nlab))
    return (logits.T, entity)
